# baseline (speedup 1.0000x reference)
import functools

import jax
import jax.numpy as jnp
from jax import lax
from jax.experimental import pallas as pl
from jax.experimental.pallas import tpu as pltpu
from jax.experimental.pallas import tpu_sc as plsc

N = 10000
E = 320000
D = 128
H = 128
C = 64
EPS = 1e-5

NC = 2
NS = 16
CH = 128
UNR = 4

N_PAD = 10112
ROWS_PER_TILE = N_PAD // NS
E_PAD = 327680
EPT = E_PAD // NS
AGG_ITERS = EPT // (UNR * CH)
DEG_ITERS = AGG_ITERS // 2

_STG_OFFS = (0, 128, 256, 384, 512)
_STG_LENS = (128, 128, 128, 128, 120)


def _mesh():
    return plsc.VectorSubcoreMesh(core_axis_name="c", subcore_axis_name="s")


_SC_PARAMS = pltpu.CompilerParams(use_tc_tiling_on_sc=False)


@functools.partial(
    pl.kernel,
    out_type=jax.ShapeDtypeStruct((NC, 2, N_PAD, 128), jnp.float32),
    mesh=_mesh(),
    compiler_params=_SC_PARAMS,
    scratch_types=[
        pltpu.VMEM_SHARED((N_PAD, 16), jnp.float32),
        pltpu.VMEM_SHARED((N_PAD, 16), jnp.float32),
        pltpu.VMEM((UNR, 2, CH), jnp.int32),
        pltpu.VMEM((CH, 16), jnp.float32),
        pltpu.VMEM((ROWS_PER_TILE, 16), jnp.float32),
    ] + [pltpu.SemaphoreType.DMA] * 8,
)
def _deg_kernel(edges_hbm, ones_hbm, zeros_hbm, out_hbm,
                acc_s, acc_d, eidx, ones_v, stage,
                s0, s1, s2, s3, s4, s5, s6, s7):
    cid = lax.axis_index("c")
    sid = lax.axis_index("s")
    wid = cid * NS + sid
    tid = wid // 2
    half = wid % 2
    r0 = sid * ROWS_PER_TILE
    pltpu.sync_copy(zeros_hbm.at[pl.ds(r0, ROWS_PER_TILE)], stage)
    pltpu.sync_copy(stage, acc_s.at[pl.ds(r0, ROWS_PER_TILE)])
    pltpu.sync_copy(stage, acc_d.at[pl.ds(r0, ROWS_PER_TILE)])
    pltpu.sync_copy(ones_hbm, ones_v)
    plsc.subcore_barrier()
    sems = (s0, s1, s2, s3, s4, s5, s6, s7)

    def body(i, carry):
        pltpu.sync_copy(edges_hbm.at[tid, half * DEG_ITERS + i], eidx)
        ds_ = []
        for j in range(UNR):
            ds_.append(pltpu.async_copy(
                ones_v, acc_s.at[eidx.at[j, 0]], sems[2 * j], add=True))
            ds_.append(pltpu.async_copy(
                ones_v, acc_d.at[eidx.at[j, 1]], sems[2 * j + 1], add=True))
        for dd in ds_:
            dd.wait()
        return carry

    lax.fori_loop(0, DEG_ITERS, body, 0)
    plsc.subcore_barrier()
    pltpu.sync_copy(acc_s.at[pl.ds(r0, ROWS_PER_TILE)], stage)
    pltpu.sync_copy(stage,
                    out_hbm.at[cid, 0, pl.ds(r0, ROWS_PER_TILE), pl.ds(0, 16)])
    pltpu.sync_copy(acc_d.at[pl.ds(r0, ROWS_PER_TILE)], stage)
    pltpu.sync_copy(stage,
                    out_hbm.at[cid, 1, pl.ds(r0, ROWS_PER_TILE), pl.ds(0, 16)])


def _make_agg_kernel(F):
    FH = F // 2

    @functools.partial(
        pl.kernel,
        out_type=jax.ShapeDtypeStruct((N_PAD, F), jnp.float32),
        mesh=_mesh(),
        compiler_params=_SC_PARAMS,
        scratch_types=[
            pltpu.VMEM_SHARED((N_PAD, FH), jnp.float32),
            pltpu.VMEM_SHARED((N_PAD, FH), jnp.float32),
            pltpu.VMEM((UNR, 2, CH), jnp.int32),
            pltpu.VMEM((UNR, 2, CH), jnp.int32),
            pltpu.VMEM((UNR, CH, FH), jnp.float32),
        ] + [pltpu.SemaphoreType.DMA] * 10,
    )
    def agg_kernel(z_hbm, edges_hbm, zeros_hbm, out_hbm,
                   z_sh, acc, eA, eB, rows,
                   g0, g1, g2, g3, s0, s1, s2, s3, iA, iB):
        cid = lax.axis_index("c")
        sid = lax.axis_index("s")
        r0 = sid * ROWS_PER_TILE
        c0 = cid * FH
        gsem = (g0, g1, g2, g3)
        ssem = (s0, s1, s2, s3)
        pltpu.sync_copy(zeros_hbm, rows.at[0])
        for off, ln in zip(_STG_OFFS, _STG_LENS):
            pltpu.sync_copy(rows.at[0, pl.ds(0, ln)],
                            acc.at[pl.ds(r0 + off, ln)])
        for off, ln in zip(_STG_OFFS, _STG_LENS):
            pltpu.sync_copy(z_hbm.at[pl.ds(r0 + off, ln), pl.ds(c0, FH)],
                            rows.at[0, pl.ds(0, ln)])
            pltpu.sync_copy(rows.at[0, pl.ds(0, ln)],
                            z_sh.at[pl.ds(r0 + off, ln)])
        plsc.subcore_barrier()

        pltpu.sync_copy(edges_hbm.at[sid, 0], eA)

        def body(i, carry):
            @pl.when(i > 0)
            def _():
                for j in range(UNR):
                    pltpu.make_async_copy(rows.at[j], acc.at[eB.at[j, 1]],
                                          ssem[j]).wait()
            gathers = [
                pltpu.async_copy(z_sh.at[eA.at[j, 0]], rows.at[j], gsem[j])
                for j in range(UNR)]
            ib = pltpu.async_copy(edges_hbm.at[sid, 2 * i + 1], eB, iB)
            scatters = []
            for j in range(UNR):
                gathers[j].wait()
                scatters.append(
                    pltpu.async_copy(rows.at[j], acc.at[eA.at[j, 1]],
                                     ssem[j], add=True))
            ib.wait()
            gathers_b = []
            for j in range(UNR):
                scatters[j].wait()
                gathers_b.append(
                    pltpu.async_copy(z_sh.at[eB.at[j, 0]], rows.at[j],
                                     gsem[j]))
            ia = pltpu.async_copy(edges_hbm.at[sid, 2 * i + 2], eA, iA)
            for j in range(UNR):
                gathers_b[j].wait()
                pltpu.async_copy(rows.at[j], acc.at[eB.at[j, 1]], ssem[j],
                                 add=True)
            ia.wait()
            return carry

        lax.fori_loop(0, AGG_ITERS // 2, body, 0)
        for j in range(UNR):
            pltpu.make_async_copy(rows.at[j], acc.at[eB.at[j, 1]],
                                  ssem[j]).wait()
        plsc.subcore_barrier()
        for off, ln in zip(_STG_OFFS, _STG_LENS):
            pltpu.sync_copy(acc.at[pl.ds(r0 + off, ln)],
                            rows.at[0, pl.ds(0, ln)])
            pltpu.sync_copy(rows.at[0, pl.ds(0, ln)],
                            out_hbm.at[pl.ds(r0 + off, ln), pl.ds(c0, FH)])

    return agg_kernel


_agg128 = _make_agg_kernel(H)
_agg64 = _make_agg_kernel(C)


def _tc_l0_body(degp, x, W0, L0, z_out, r_out, ms_out, md_out):
    dsrc = degp[0, 0, :, 0:1] + degp[1, 0, :, 0:1]
    ddst = degp[0, 1, :, 0:1] + degp[1, 1, :, 0:1]
    inv_s = lax.rsqrt(jnp.maximum(dsrc, 1.0))
    inv_d = lax.rsqrt(jnp.maximum(ddst, 1.0))
    ms = jnp.broadcast_to(inv_s, (N_PAD, H))
    md = jnp.broadcast_to(inv_d, (N_PAD, H))
    ms_out[...] = ms
    md_out[...] = md
    z_out[...] = jnp.dot(x[...] * ms, W0[...],
                         preferred_element_type=jnp.float32)
    r_out[...] = jnp.dot(x[...], L0[...], preferred_element_type=jnp.float32)


def _tc_mid_body(agg, r_prev, md, ms, b, g, be, W, L, z_out, r_out):
    t = agg[...] * md[...] + b[...] + r_prev[...]
    mask = (lax.broadcasted_iota(jnp.int32, (N_PAD, 1), 0) < N).astype(
        jnp.float32)
    cnt = float(N)
    mu = jnp.sum(t * mask, axis=0, keepdims=True) / cnt
    dd = (t - mu) * mask
    var = jnp.sum(dd * dd, axis=0, keepdims=True) / cnt
    hn = (t - mu) * lax.rsqrt(var + EPS) * g[...] + be[...]
    h = jnp.maximum(hn, 0.0) * mask
    z_out[...] = jnp.dot(h * ms[...], W[...],
                         preferred_element_type=jnp.float32)
    r_out[...] = jnp.dot(h, L[...], preferred_element_type=jnp.float32)


def _tc_fin_body(agg, r2, md, b, g, be, out):
    t = agg[...] * md[:, :C] + b[...] + r2[...]
    tv = t[:N]
    mu = jnp.sum(tv, axis=0, keepdims=True) / float(N)
    dd = tv - mu
    var = jnp.sum(dd * dd, axis=0, keepdims=True) / float(N)
    h = jnp.maximum(dd * lax.rsqrt(var + EPS) * g[...] + be[...], 0.0)
    m = jnp.max(h, axis=1, keepdims=True)
    e = jnp.exp(h - m)
    out[...] = e / jnp.sum(e, axis=1, keepdims=True)


def _f32(shape):
    return jax.ShapeDtypeStruct(shape, jnp.float32)


def kernel(x, edge_index, W0, b0, L0, g0, be0, W1, b1, L1, g1, be1,
           W2, b2, L2, g2, be2):
    f32 = jnp.float32
    x_pad = jnp.zeros((N_PAD, D), f32).at[:N, :].set(x)
    src = edge_index[0]
    dst = edge_index[1]
    pad_n = E_PAD - E
    pad_idx = (N + (jnp.arange(pad_n, dtype=jnp.int32) % 16)).astype(jnp.int32)
    src_p = jnp.concatenate([src, pad_idx])
    dst_p = jnp.concatenate([dst, pad_idx])
    e_agg = jnp.stack(
        [src_p.reshape(NS, AGG_ITERS, UNR, CH),
         dst_p.reshape(NS, AGG_ITERS, UNR, CH)], axis=3)
    e_agg = jnp.concatenate(
        [e_agg, jnp.full((NS, 1, UNR, 2, CH), N, jnp.int32)], axis=1)
    ones16 = jnp.ones((CH, 16), f32)
    zeros16 = jnp.zeros((N_PAD, 16), f32)
    zeros64 = jnp.zeros((CH, H // 2), f32)
    zeros32 = jnp.zeros((CH, C // 2), f32)
    b0r, g0r, be0r = b0.reshape(1, H), g0.reshape(1, H), be0.reshape(1, H)
    b1r, g1r, be1r = b1.reshape(1, H), g1.reshape(1, H), be1.reshape(1, H)
    b2r, g2r, be2r = b2.reshape(1, C), g2.reshape(1, C), be2.reshape(1, C)

    degp = _deg_kernel(e_agg, ones16, zeros16)

    z0, r0, ms, md = pl.pallas_call(
        _tc_l0_body,
        out_shape=[_f32((N_PAD, H)), _f32((N_PAD, H)),
                   _f32((N_PAD, H)), _f32((N_PAD, H))],
    )(degp, x_pad, W0, L0)

    agg0 = _agg128(z0, e_agg, zeros64)

    z1, r1 = pl.pallas_call(
        _tc_mid_body,
        out_shape=[_f32((N_PAD, H)), _f32((N_PAD, H))],
    )(agg0, r0, md, ms, b0r, g0r, be0r, W1, L1)

    agg1 = _agg128(z1, e_agg, zeros64)

    z2, r2 = pl.pallas_call(
        _tc_mid_body,
        out_shape=[_f32((N_PAD, C)), _f32((N_PAD, C))],
    )(agg1, r1, md, ms, b1r, g1r, be1r, W2, L2)

    agg2 = _agg64(z2, e_agg, zeros32)

    out = pl.pallas_call(
        _tc_fin_body,
        out_shape=_f32((N, C)),
    )(agg2, r2, md, b2r, g2r, be2r)

    return out


# speedup vs baseline: 1.2653x; 1.2653x over previous
import functools

import jax
import jax.numpy as jnp
from jax import lax
from jax.experimental import pallas as pl
from jax.experimental.pallas import tpu as pltpu
from jax.experimental.pallas import tpu_sc as plsc

N = 10000
E = 320000
D = 128
H = 128
C = 64
EPS = 1e-5

NC = 2
NS = 16
CH = 128
UNR = 4

N_PAD = 10112
ROWS_PER_TILE = N_PAD // NS
E_PAD = 327680
EPT = E_PAD // NS
AGG_ITERS = EPT // (UNR * CH)
DEG_ITERS = AGG_ITERS // 2

_STG_OFFS = (0, 128, 256, 384, 512)
_STG_LENS = (128, 128, 128, 128, 120)


def _mesh():
    return plsc.VectorSubcoreMesh(core_axis_name="c", subcore_axis_name="s")


_SC_PARAMS = pltpu.CompilerParams(use_tc_tiling_on_sc=False)


@functools.partial(
    pl.kernel,
    out_type=jax.ShapeDtypeStruct((NC, 2, N_PAD, 128), jnp.float32),
    mesh=_mesh(),
    compiler_params=_SC_PARAMS,
    scratch_types=[
        pltpu.VMEM_SHARED((N_PAD, 16), jnp.float32),
        pltpu.VMEM_SHARED((N_PAD, 16), jnp.float32),
        pltpu.VMEM((UNR, 2, CH), jnp.int32),
        pltpu.VMEM((CH, 16), jnp.float32),
        pltpu.VMEM((ROWS_PER_TILE, 16), jnp.float32),
    ] + [pltpu.SemaphoreType.DMA] * 8,
)
def _deg_kernel(edges_hbm, ones_hbm, zeros_hbm, out_hbm,
                acc_s, acc_d, eidx, ones_v, stage,
                s0, s1, s2, s3, s4, s5, s6, s7):
    cid = lax.axis_index("c")
    sid = lax.axis_index("s")
    wid = cid * NS + sid
    tid = wid // 2
    half = wid % 2
    r0 = sid * ROWS_PER_TILE
    pltpu.sync_copy(zeros_hbm.at[pl.ds(r0, ROWS_PER_TILE)], stage)
    pltpu.sync_copy(stage, acc_s.at[pl.ds(r0, ROWS_PER_TILE)])
    pltpu.sync_copy(stage, acc_d.at[pl.ds(r0, ROWS_PER_TILE)])
    pltpu.sync_copy(ones_hbm, ones_v)
    plsc.subcore_barrier()
    sems = (s0, s1, s2, s3, s4, s5, s6, s7)

    def body(i, carry):
        pltpu.sync_copy(edges_hbm.at[tid, half * DEG_ITERS + i], eidx)
        ds_ = []
        for j in range(UNR):
            ds_.append(pltpu.async_copy(
                ones_v, acc_s.at[eidx.at[j, 0]], sems[2 * j], add=True))
            ds_.append(pltpu.async_copy(
                ones_v, acc_d.at[eidx.at[j, 1]], sems[2 * j + 1], add=True))
        for dd in ds_:
            dd.wait()
        return carry

    lax.fori_loop(0, DEG_ITERS, body, 0)
    plsc.subcore_barrier()
    pltpu.sync_copy(acc_s.at[pl.ds(r0, ROWS_PER_TILE)], stage)
    pltpu.sync_copy(stage,
                    out_hbm.at[cid, 0, pl.ds(r0, ROWS_PER_TILE), pl.ds(0, 16)])
    pltpu.sync_copy(acc_d.at[pl.ds(r0, ROWS_PER_TILE)], stage)
    pltpu.sync_copy(stage,
                    out_hbm.at[cid, 1, pl.ds(r0, ROWS_PER_TILE), pl.ds(0, 16)])


def _make_agg_kernel(F):
    FH = F // 2

    @functools.partial(
        pl.kernel,
        out_type=jax.ShapeDtypeStruct((N_PAD, F), jnp.float32),
        mesh=_mesh(),
        compiler_params=_SC_PARAMS,
        scratch_types=[
            pltpu.VMEM_SHARED((N_PAD, FH), jnp.float32),
            pltpu.VMEM_SHARED((N_PAD, FH), jnp.float32),
            pltpu.VMEM((UNR, 2, CH), jnp.int32),
            pltpu.VMEM((UNR, 2, CH), jnp.int32),
            pltpu.VMEM((UNR, CH, FH), jnp.float32),
        ] + [pltpu.SemaphoreType.DMA] * 10,
    )
    def agg_kernel(z_hbm, edges_hbm, zeros_hbm, out_hbm,
                   z_sh, acc, eA, eB, rows,
                   g0, g1, g2, g3, s0, s1, s2, s3, iA, iB):
        cid = lax.axis_index("c")
        sid = lax.axis_index("s")
        r0 = sid * ROWS_PER_TILE
        c0 = cid * FH
        gsem = (g0, g1, g2, g3)
        ssem = (s0, s1, s2, s3)
        pltpu.sync_copy(zeros_hbm, rows.at[0])
        for off, ln in zip(_STG_OFFS, _STG_LENS):
            pltpu.sync_copy(rows.at[0, pl.ds(0, ln)],
                            acc.at[pl.ds(r0 + off, ln)])
        for off, ln in zip(_STG_OFFS, _STG_LENS):
            pltpu.sync_copy(z_hbm.at[pl.ds(r0 + off, ln), pl.ds(c0, FH)],
                            rows.at[0, pl.ds(0, ln)])
            pltpu.sync_copy(rows.at[0, pl.ds(0, ln)],
                            z_sh.at[pl.ds(r0 + off, ln)])
        plsc.subcore_barrier()

        pltpu.sync_copy(edges_hbm.at[sid, 0], eA)

        def block(eidx):
            gathers = [
                pltpu.async_copy(z_sh.at[eidx.at[j, 0]], rows.at[j], gsem[j])
                for j in range(UNR)]
            scatters = []
            for j in range(UNR):
                gathers[j].wait()
                scatters.append(
                    pltpu.async_copy(rows.at[j], acc.at[eidx.at[j, 1]],
                                     ssem[j], add=True))
            for sc in scatters:
                sc.wait()

        def body(i, carry):
            ib = pltpu.async_copy(edges_hbm.at[sid, 2 * i + 1], eB, iB)
            block(eA)
            ib.wait()
            ia = pltpu.async_copy(edges_hbm.at[sid, 2 * i + 2], eA, iA)
            block(eB)
            ia.wait()
            return carry

        lax.fori_loop(0, AGG_ITERS // 2, body, 0)
        plsc.subcore_barrier()
        for off, ln in zip(_STG_OFFS, _STG_LENS):
            pltpu.sync_copy(acc.at[pl.ds(r0 + off, ln)],
                            rows.at[0, pl.ds(0, ln)])
            pltpu.sync_copy(rows.at[0, pl.ds(0, ln)],
                            out_hbm.at[pl.ds(r0 + off, ln), pl.ds(c0, FH)])

    return agg_kernel


_agg128 = _make_agg_kernel(H)
_agg64 = _make_agg_kernel(C)


def _tc_l0_body(degp, x, W0, L0, z_out, r_out, ms_out, md_out):
    dsrc = degp[0, 0, :, 0:1] + degp[1, 0, :, 0:1]
    ddst = degp[0, 1, :, 0:1] + degp[1, 1, :, 0:1]
    inv_s = lax.rsqrt(jnp.maximum(dsrc, 1.0))
    inv_d = lax.rsqrt(jnp.maximum(ddst, 1.0))
    ms = jnp.broadcast_to(inv_s, (N_PAD, H))
    md = jnp.broadcast_to(inv_d, (N_PAD, H))
    ms_out[...] = ms
    md_out[...] = md
    z_out[...] = jnp.dot(x[...] * ms, W0[...],
                         preferred_element_type=jnp.float32)
    r_out[...] = jnp.dot(x[...], L0[...], preferred_element_type=jnp.float32)


def _tc_mid_body(agg, r_prev, md, ms, b, g, be, W, L, z_out, r_out):
    t = agg[...] * md[...] + b[...] + r_prev[...]
    mask = (lax.broadcasted_iota(jnp.int32, (N_PAD, 1), 0) < N).astype(
        jnp.float32)
    cnt = float(N)
    mu = jnp.sum(t * mask, axis=0, keepdims=True) / cnt
    dd = (t - mu) * mask
    var = jnp.sum(dd * dd, axis=0, keepdims=True) / cnt
    hn = (t - mu) * lax.rsqrt(var + EPS) * g[...] + be[...]
    h = jnp.maximum(hn, 0.0) * mask
    z_out[...] = jnp.dot(h * ms[...], W[...],
                         preferred_element_type=jnp.float32)
    r_out[...] = jnp.dot(h, L[...], preferred_element_type=jnp.float32)


def _tc_fin_body(agg, r2, md, b, g, be, out):
    t = agg[...] * md[:, :C] + b[...] + r2[...]
    tv = t[:N]
    mu = jnp.sum(tv, axis=0, keepdims=True) / float(N)
    dd = tv - mu
    var = jnp.sum(dd * dd, axis=0, keepdims=True) / float(N)
    h = jnp.maximum(dd * lax.rsqrt(var + EPS) * g[...] + be[...], 0.0)
    m = jnp.max(h, axis=1, keepdims=True)
    e = jnp.exp(h - m)
    out[...] = e / jnp.sum(e, axis=1, keepdims=True)


def _f32(shape):
    return jax.ShapeDtypeStruct(shape, jnp.float32)


def kernel(x, edge_index, W0, b0, L0, g0, be0, W1, b1, L1, g1, be1,
           W2, b2, L2, g2, be2):
    f32 = jnp.float32
    x_pad = jnp.zeros((N_PAD, D), f32).at[:N, :].set(x)
    src = edge_index[0]
    dst = edge_index[1]
    pad_n = E_PAD - E
    pad_idx = (N + (jnp.arange(pad_n, dtype=jnp.int32) % 16)).astype(jnp.int32)
    src_p = jnp.concatenate([src, pad_idx])
    dst_p = jnp.concatenate([dst, pad_idx])
    e_agg = jnp.stack(
        [src_p.reshape(NS, AGG_ITERS, UNR, CH),
         dst_p.reshape(NS, AGG_ITERS, UNR, CH)], axis=3)
    e_agg = jnp.concatenate(
        [e_agg, jnp.full((NS, 1, UNR, 2, CH), N, jnp.int32)], axis=1)
    ones16 = jnp.ones((CH, 16), f32)
    zeros16 = jnp.zeros((N_PAD, 16), f32)
    zeros64 = jnp.zeros((CH, H // 2), f32)
    zeros32 = jnp.zeros((CH, C // 2), f32)
    b0r, g0r, be0r = b0.reshape(1, H), g0.reshape(1, H), be0.reshape(1, H)
    b1r, g1r, be1r = b1.reshape(1, H), g1.reshape(1, H), be1.reshape(1, H)
    b2r, g2r, be2r = b2.reshape(1, C), g2.reshape(1, C), be2.reshape(1, C)

    degp = _deg_kernel(e_agg, ones16, zeros16)

    z0, r0, ms, md = pl.pallas_call(
        _tc_l0_body,
        out_shape=[_f32((N_PAD, H)), _f32((N_PAD, H)),
                   _f32((N_PAD, H)), _f32((N_PAD, H))],
    )(degp, x_pad, W0, L0)

    agg0 = _agg128(z0, e_agg, zeros64)

    z1, r1 = pl.pallas_call(
        _tc_mid_body,
        out_shape=[_f32((N_PAD, H)), _f32((N_PAD, H))],
    )(agg0, r0, md, ms, b0r, g0r, be0r, W1, L1)

    agg1 = _agg128(z1, e_agg, zeros64)

    z2, r2 = pl.pallas_call(
        _tc_mid_body,
        out_shape=[_f32((N_PAD, C)), _f32((N_PAD, C))],
    )(agg1, r1, md, ms, b1r, g1r, be1r, W2, L2)

    agg2 = _agg64(z2, e_agg, zeros32)

    out = pl.pallas_call(
        _tc_fin_body,
        out_shape=_f32((N, C)),
    )(agg2, r2, md, b2r, g2r, be2r)

    return out


# speedup vs baseline: 1.2924x; 1.0214x over previous
import functools

import jax
import jax.numpy as jnp
from jax import lax
from jax.experimental import pallas as pl
from jax.experimental.pallas import tpu as pltpu
from jax.experimental.pallas import tpu_sc as plsc

N = 10000
E = 320000
D = 128
H = 128
C = 64
EPS = 1e-5

NC = 2
NS = 16
CH = 128
UNR = 4

N_PAD = 10112
ROWS_PER_TILE = N_PAD // NS
E_PAD = 327680
EPT = E_PAD // NS
AGG_ITERS = EPT // (UNR * CH)
DEG_ITERS = AGG_ITERS // 2

_STG_OFFS = (0, 128, 256, 384, 512)
_STG_LENS = (128, 128, 128, 128, 120)


def _mesh():
    return plsc.VectorSubcoreMesh(core_axis_name="c", subcore_axis_name="s")


_SC_PARAMS = pltpu.CompilerParams(use_tc_tiling_on_sc=False)


@functools.partial(
    pl.kernel,
    out_type=jax.ShapeDtypeStruct((NC, 2, N_PAD, 128), jnp.float32),
    mesh=_mesh(),
    compiler_params=_SC_PARAMS,
    scratch_types=[
        pltpu.VMEM_SHARED((N_PAD, 16), jnp.float32),
        pltpu.VMEM_SHARED((N_PAD, 16), jnp.float32),
        pltpu.VMEM((UNR, 2, CH), jnp.int32),
        pltpu.VMEM((UNR, 2, CH), jnp.int32),
        pltpu.VMEM((CH, 16), jnp.float32),
        pltpu.VMEM((ROWS_PER_TILE, 16), jnp.float32),
    ] + [pltpu.SemaphoreType.DMA] * 10,
)
def _deg_kernel(edges_hbm, ones_hbm, zeros_hbm, out_hbm,
                acc_s, acc_d, eA, eB, ones_v, stage,
                s0, s1, s2, s3, s4, s5, s6, s7, iA, iB):
    cid = lax.axis_index("c")
    sid = lax.axis_index("s")
    wid = cid * NS + sid
    tid = wid // 2
    half = wid % 2
    r0 = sid * ROWS_PER_TILE
    pltpu.sync_copy(zeros_hbm.at[pl.ds(r0, ROWS_PER_TILE)], stage)
    pltpu.sync_copy(stage, acc_s.at[pl.ds(r0, ROWS_PER_TILE)])
    pltpu.sync_copy(stage, acc_d.at[pl.ds(r0, ROWS_PER_TILE)])
    pltpu.sync_copy(ones_hbm, ones_v)
    plsc.subcore_barrier()
    sems = (s0, s1, s2, s3, s4, s5, s6, s7)
    b0 = half * DEG_ITERS
    pltpu.sync_copy(edges_hbm.at[tid, b0], eA)

    def block(eidx):
        ds_ = []
        for j in range(UNR):
            ds_.append(pltpu.async_copy(
                ones_v, acc_s.at[eidx.at[j, 0]], sems[2 * j], add=True))
            ds_.append(pltpu.async_copy(
                ones_v, acc_d.at[eidx.at[j, 1]], sems[2 * j + 1], add=True))
        for dd in ds_:
            dd.wait()

    def body(i, carry):
        ib = pltpu.async_copy(edges_hbm.at[tid, b0 + 2 * i + 1], eB, iB)
        block(eA)
        ib.wait()
        ia = pltpu.async_copy(edges_hbm.at[tid, b0 + 2 * i + 2], eA, iA)
        block(eB)
        ia.wait()
        return carry

    lax.fori_loop(0, DEG_ITERS // 2, body, 0)
    plsc.subcore_barrier()
    pltpu.sync_copy(acc_s.at[pl.ds(r0, ROWS_PER_TILE)], stage)
    pltpu.sync_copy(stage,
                    out_hbm.at[cid, 0, pl.ds(r0, ROWS_PER_TILE), pl.ds(0, 16)])
    pltpu.sync_copy(acc_d.at[pl.ds(r0, ROWS_PER_TILE)], stage)
    pltpu.sync_copy(stage,
                    out_hbm.at[cid, 1, pl.ds(r0, ROWS_PER_TILE), pl.ds(0, 16)])


def _make_agg_kernel(F):
    FH = F // 2

    @functools.partial(
        pl.kernel,
        out_type=jax.ShapeDtypeStruct((N_PAD, F), jnp.float32),
        mesh=_mesh(),
        compiler_params=_SC_PARAMS,
        scratch_types=[
            pltpu.VMEM_SHARED((N_PAD, FH), jnp.float32),
            pltpu.VMEM_SHARED((N_PAD, FH), jnp.float32),
            pltpu.VMEM((UNR, 2, CH), jnp.int32),
            pltpu.VMEM((UNR, 2, CH), jnp.int32),
            pltpu.VMEM((UNR, CH, FH), jnp.float32),
        ] + [pltpu.SemaphoreType.DMA] * 10,
    )
    def agg_kernel(z_hbm, edges_hbm, zeros_hbm, out_hbm,
                   z_sh, acc, eA, eB, rows,
                   g0, g1, g2, g3, s0, s1, s2, s3, iA, iB):
        cid = lax.axis_index("c")
        sid = lax.axis_index("s")
        r0 = sid * ROWS_PER_TILE
        c0 = cid * FH
        gsem = (g0, g1, g2, g3)
        ssem = (s0, s1, s2, s3)
        pltpu.sync_copy(zeros_hbm, rows.at[0])
        for off, ln in zip(_STG_OFFS, _STG_LENS):
            pltpu.sync_copy(rows.at[0, pl.ds(0, ln)],
                            acc.at[pl.ds(r0 + off, ln)])
        for off, ln in zip(_STG_OFFS, _STG_LENS):
            pltpu.sync_copy(z_hbm.at[pl.ds(r0 + off, ln), pl.ds(c0, FH)],
                            rows.at[0, pl.ds(0, ln)])
            pltpu.sync_copy(rows.at[0, pl.ds(0, ln)],
                            z_sh.at[pl.ds(r0 + off, ln)])
        plsc.subcore_barrier()

        pltpu.sync_copy(edges_hbm.at[sid, 0], eA)

        def block(eidx):
            gathers = [
                pltpu.async_copy(z_sh.at[eidx.at[j, 0]], rows.at[j], gsem[j])
                for j in range(UNR)]
            scatters = []
            for j in range(UNR):
                gathers[j].wait()
                scatters.append(
                    pltpu.async_copy(rows.at[j], acc.at[eidx.at[j, 1]],
                                     ssem[j], add=True))
            for sc in scatters:
                sc.wait()

        def body(i, carry):
            ib = pltpu.async_copy(edges_hbm.at[sid, 2 * i + 1], eB, iB)
            block(eA)
            ib.wait()
            ia = pltpu.async_copy(edges_hbm.at[sid, 2 * i + 2], eA, iA)
            block(eB)
            ia.wait()
            return carry

        lax.fori_loop(0, AGG_ITERS // 2, body, 0)
        plsc.subcore_barrier()
        for off, ln in zip(_STG_OFFS, _STG_LENS):
            pltpu.sync_copy(acc.at[pl.ds(r0 + off, ln)],
                            rows.at[0, pl.ds(0, ln)])
            pltpu.sync_copy(rows.at[0, pl.ds(0, ln)],
                            out_hbm.at[pl.ds(r0 + off, ln), pl.ds(c0, FH)])

    return agg_kernel


_agg128 = _make_agg_kernel(H)
_agg64 = _make_agg_kernel(C)


def _tc_l0_body(degp, x, W0, L0, z_out, r_out, ms_out, md_out):
    dsrc = degp[0, 0, :, 0:1] + degp[1, 0, :, 0:1]
    ddst = degp[0, 1, :, 0:1] + degp[1, 1, :, 0:1]
    inv_s = lax.rsqrt(jnp.maximum(dsrc, 1.0))
    inv_d = lax.rsqrt(jnp.maximum(ddst, 1.0))
    ms = jnp.broadcast_to(inv_s, (N_PAD, H))
    md = jnp.broadcast_to(inv_d, (N_PAD, H))
    ms_out[...] = ms
    md_out[...] = md
    z_out[...] = jnp.dot(x[...] * ms, W0[...],
                         preferred_element_type=jnp.float32)
    r_out[...] = jnp.dot(x[...], L0[...], preferred_element_type=jnp.float32)


def _tc_mid_body(agg, r_prev, md, ms, b, g, be, W, L, z_out, r_out):
    t = agg[...] * md[...] + b[...] + r_prev[...]
    mask = (lax.broadcasted_iota(jnp.int32, (N_PAD, 1), 0) < N).astype(
        jnp.float32)
    cnt = float(N)
    mu = jnp.sum(t * mask, axis=0, keepdims=True) / cnt
    dd = (t - mu) * mask
    var = jnp.sum(dd * dd, axis=0, keepdims=True) / cnt
    hn = (t - mu) * lax.rsqrt(var + EPS) * g[...] + be[...]
    h = jnp.maximum(hn, 0.0) * mask
    z_out[...] = jnp.dot(h * ms[...], W[...],
                         preferred_element_type=jnp.float32)
    r_out[...] = jnp.dot(h, L[...], preferred_element_type=jnp.float32)


def _tc_fin_body(agg, r2, md, b, g, be, out):
    t = agg[...] * md[:, :C] + b[...] + r2[...]
    tv = t[:N]
    mu = jnp.sum(tv, axis=0, keepdims=True) / float(N)
    dd = tv - mu
    var = jnp.sum(dd * dd, axis=0, keepdims=True) / float(N)
    h = jnp.maximum(dd * lax.rsqrt(var + EPS) * g[...] + be[...], 0.0)
    m = jnp.max(h, axis=1, keepdims=True)
    e = jnp.exp(h - m)
    out[...] = e / jnp.sum(e, axis=1, keepdims=True)


def _f32(shape):
    return jax.ShapeDtypeStruct(shape, jnp.float32)


def kernel(x, edge_index, W0, b0, L0, g0, be0, W1, b1, L1, g1, be1,
           W2, b2, L2, g2, be2):
    f32 = jnp.float32
    x_pad = jnp.zeros((N_PAD, D), f32).at[:N, :].set(x)
    src = edge_index[0]
    dst = edge_index[1]
    pad_n = E_PAD - E
    pad_idx = (N + (jnp.arange(pad_n, dtype=jnp.int32) % 16)).astype(jnp.int32)
    src_p = jnp.concatenate([src, pad_idx])
    dst_p = jnp.concatenate([dst, pad_idx])
    e_agg = jnp.stack(
        [src_p.reshape(NS, AGG_ITERS, UNR, CH),
         dst_p.reshape(NS, AGG_ITERS, UNR, CH)], axis=3)
    e_agg = jnp.concatenate(
        [e_agg, jnp.full((NS, 1, UNR, 2, CH), N, jnp.int32)], axis=1)
    ones16 = jnp.ones((CH, 16), f32)
    zeros16 = jnp.zeros((N_PAD, 16), f32)
    zeros64 = jnp.zeros((CH, H // 2), f32)
    zeros32 = jnp.zeros((CH, C // 2), f32)
    b0r, g0r, be0r = b0.reshape(1, H), g0.reshape(1, H), be0.reshape(1, H)
    b1r, g1r, be1r = b1.reshape(1, H), g1.reshape(1, H), be1.reshape(1, H)
    b2r, g2r, be2r = b2.reshape(1, C), g2.reshape(1, C), be2.reshape(1, C)

    degp = _deg_kernel(e_agg, ones16, zeros16)

    z0, r0, ms, md = pl.pallas_call(
        _tc_l0_body,
        out_shape=[_f32((N_PAD, H)), _f32((N_PAD, H)),
                   _f32((N_PAD, H)), _f32((N_PAD, H))],
    )(degp, x_pad, W0, L0)

    agg0 = _agg128(z0, e_agg, zeros64)

    z1, r1 = pl.pallas_call(
        _tc_mid_body,
        out_shape=[_f32((N_PAD, H)), _f32((N_PAD, H))],
    )(agg0, r0, md, ms, b0r, g0r, be0r, W1, L1)

    agg1 = _agg128(z1, e_agg, zeros64)

    z2, r2 = pl.pallas_call(
        _tc_mid_body,
        out_shape=[_f32((N_PAD, C)), _f32((N_PAD, C))],
    )(agg1, r1, md, ms, b1r, g1r, be1r, W2, L2)

    agg2 = _agg64(z2, e_agg, zeros32)

    out = pl.pallas_call(
        _tc_fin_body,
        out_shape=_f32((N, C)),
    )(agg2, r2, md, b2r, g2r, be2r)

    return out
